# baseline (speedup 1.0000x reference)
import jax
import jax.numpy as jnp
from jax.experimental import pallas as pl
from jax.experimental.pallas import tpu as pltpu

_SHIFT = 50000.0
_SEQ = 200
_D = 64
_TPB = 20
_RPB = 24

_N0 = 10
_K0 = 1280
_N1 = 4
_P1 = _N0
_P2 = _P1 + _N1
_STEPS = _P2 + _N1


def _bf16(v):
    return v.astype(jnp.bfloat16)


def _fused(xs, emb, W0, b0, W1, b1, W2, b2):
    def body(xs_ref, emb_ref, w0_ref, w1_ref, w2_ref, b0_ref, b1_ref, b2_ref,
             o_ref, h0s, drain, acc0, acc1, h1r, h2r, gsem0, gsem1):
        i = pl.program_id(0)

        def issue_batch(b, gs):
            for j in range(_TPB):
                t = b * _TPB + j
                idx = (xs_ref[0, t] + _SHIFT).astype(jnp.int32)
                pltpu.make_async_copy(
                    emb_ref.at[pl.ds(idx, 1), :],
                    h0s.at[pl.ds(b * _RPB + j, 1), :],
                    gs,
                ).start()

        def wait_batch(gs):
            pltpu.make_async_copy(
                emb_ref.at[pl.ds(0, _TPB), :], drain, gs
            ).wait()

        @pl.when(i == 0)
        def _():
            acc0[...] = jnp.zeros_like(acc0)
            acc1[...] = jnp.zeros_like(acc1)
            issue_batch(0, gsem0)

        @pl.when((i < _N0 - 1) & ((i % 2) == 0))
        def _():
            issue_batch(i + 1, gsem1)

        @pl.when((i < _N0 - 1) & ((i % 2) == 1))
        def _():
            issue_batch(i + 1, gsem0)

        @pl.when((i < _N0) & ((i % 2) == 0))
        def _():
            wait_batch(gsem0)

        @pl.when((i < _N0) & ((i % 2) == 1))
        def _():
            wait_batch(gsem1)

        @pl.when(i < _N0)
        def _():
            hv = _bf16(h0s[pl.ds(i * _RPB, _RPB), :])
            wb = _bf16(w0_ref[...])
            r = acc0[...]
            for j in range(_TPB):
                r += jnp.dot(
                    hv[j:j + 1, :], wb[j * _D:(j + 1) * _D, :],
                    preferred_element_type=jnp.float32,
                )
            acc0[...] = r

        @pl.when(i == _P1 - 1)
        def _():
            r = jnp.tanh(acc0[...] + b0_ref[...])
            for q in range(_N1):
                h1r[q:q + 1, :] = r[:, q * 512:(q + 1) * 512]

        @pl.when((i >= _P1) & (i < _P2))
        def _():
            k = i - _P1
            acc1[...] += jnp.dot(
                _bf16(h1r[pl.ds(k, 1), :]), _bf16(w1_ref[...]),
                preferred_element_type=jnp.float32,
            )

        @pl.when(i == _P2 - 1)
        def _():
            r = jnp.tanh(acc1[...] + b1_ref[...])
            for q in range(_N1):
                h2r[q:q + 1, :] = r[:, q * 512:(q + 1) * 512]

        @pl.when(i >= _P2)
        def _():
            k = i - _P2

            @pl.when(k == 0)
            def _():
                o_ref[...] = b2_ref[...]

            o_ref[...] += jnp.dot(
                _bf16(h2r[pl.ds(k, 1), :]), _bf16(w2_ref[...]),
                preferred_element_type=jnp.float32,
            )

    c0 = lambda i: jnp.minimum(i, _N0 - 1)
    c1 = lambda i: jnp.clip(i - _P1, 0, _N1 - 1)
    c2 = lambda i: jnp.clip(i - _P2, 0, _N1 - 1)

    return pl.pallas_call(
        body,
        grid=(_STEPS,),
        in_specs=[
            pl.BlockSpec(memory_space=pltpu.MemorySpace.SMEM),
            pl.BlockSpec(memory_space=pltpu.MemorySpace.HBM),
            pl.BlockSpec((_K0, 2048), lambda i: (c0(i), 0)),
            pl.BlockSpec((512, 2048), lambda i: (c1(i), 0)),
            pl.BlockSpec((512, 2048), lambda i: (c2(i), 0)),
            pl.BlockSpec((1, 2048), lambda i: (0, 0)),
            pl.BlockSpec((1, 2048), lambda i: (0, 0)),
            pl.BlockSpec((1, 2048), lambda i: (0, 0)),
        ],
        out_specs=pl.BlockSpec((1, 2048), lambda i: (0, 0)),
        out_shape=jax.ShapeDtypeStruct((1, 2048), jnp.float32),
        scratch_shapes=[
            pltpu.VMEM((_N0 * _RPB, _D), jnp.float32),
            pltpu.VMEM((_TPB, _D), jnp.float32),
            pltpu.VMEM((1, 2048), jnp.float32),
            pltpu.VMEM((1, 2048), jnp.float32),
            pltpu.VMEM((_N1, 512), jnp.float32),
            pltpu.VMEM((_N1, 512), jnp.float32),
            pltpu.SemaphoreType.DMA,
            pltpu.SemaphoreType.DMA,
        ],
    )(xs, emb, W0, W1, W2, b0, b1, b2)


def kernel(x, embedding, W0, b0, W1, b1, W2, b2):
    out = _fused(
        x.reshape(1, _SEQ), embedding, W0, b0.reshape(1, -1),
        W1, b1.reshape(1, -1), W2, b2.reshape(1, -1),
    )
    return out.reshape(-1)
